# baseline (speedup 1.0000x reference)
import functools
import itertools

import jax
import jax.numpy as jnp
from jax import lax
from jax.experimental import pallas as pl
from jax.experimental.pallas import tpu as pltpu
from jax.experimental.pallas import tpu_sc as plsc

P0, P1, P2 = 73856093, 19349663, 83492791
NC, NS, L = 2, 16, 16
VD1 = 9
TR = 4096


@functools.cache
def _build(n, m, ch):
    nw = NC * NS
    sweeps = m // (TR * nw)
    assert sweeps * TR * nw == m
    totchunks = n // ch
    groups = ch // L

    def body(lat_hbm, pos_hbm, hom_hbm, out_hbm, pos_v, hom_v, accum):
        c = lax.axis_index("c")
        s = lax.axis_index("s")
        wid = s * NC + c

        def do_sweep(j, carry):
            base = (j * nw + wid) * TR
            pltpu.sync_copy(lat_hbm.at[pl.ds(base * VD1, TR * VD1)], accum)

            def do_chunk(k, carry):
                pltpu.sync_copy(pos_hbm.at[k], pos_v)
                pltpu.sync_copy(hom_hbm.at[k], hom_v)

                def do_group(g, carry):
                    col = g * L

                    def flo(f):
                        i = f.astype(jnp.int32)
                        i = jnp.where(f < i.astype(jnp.float32), i - 1, i)
                        return i, f - i.astype(jnp.float32)

                    bx, fx = flo(pos_v[0, pl.ds(col, L)])
                    by, fy = flo(pos_v[1, pl.ds(col, L)])
                    bz, fz = flo(pos_v[2, pl.ds(col, L)])
                    h0 = bx * P0 + by * P1 + bz * P2
                    wx0 = 1.0 - fx
                    wy0 = 1.0 - fy
                    wz0 = 1.0 - fz
                    wxy = (wx0 * wy0, wx0 * fy, fx * wy0, fx * fy)
                    v = [hom_v[r, pl.ds(col, L)] for r in range(VD1)]
                    basev = jnp.full((L,), base, jnp.int32)
                    for ci, (ox, oy, oz) in enumerate(
                        itertools.product((0, 1), repeat=3)
                    ):
                        cc = (ox * P0 + oy * P1 + oz * P2) & 0xFFFFFFFF
                        cc = cc - (1 << 32) if cc >= (1 << 31) else cc
                        hc = (h0 + cc) & (m - 1)
                        loc = hc - basev
                        inp = (loc >= jnp.zeros((L,), jnp.int32)) & (
                            loc < jnp.full((L,), TR, jnp.int32)
                        )
                        locs = (loc & (TR - 1)) * VD1
                        wc = wxy[ox * 2 + oy] * (fz if oz else wz0)
                        we = jnp.where(inp, wc, 0.0)
                        for r in range(VD1):
                            plsc.addupdate_scatter(
                                accum, [locs + r], we * v[r]
                            )
                    return carry

                return lax.fori_loop(0, groups, do_group, carry)

            carry = lax.fori_loop(0, totchunks, do_chunk, carry)
            pltpu.sync_copy(accum, out_hbm.at[pl.ds(base * VD1, TR * VD1)])
            return carry

        lax.fori_loop(0, sweeps, do_sweep, 0)

    mesh = plsc.VectorSubcoreMesh(
        core_axis_name="c", subcore_axis_name="s", num_cores=NC, num_subcores=NS
    )
    return pl.kernel(
        body,
        out_type=jax.ShapeDtypeStruct((m * VD1,), jnp.float32),
        mesh=mesh,
        compiler_params=pltpu.CompilerParams(needs_layout_passes=False),
        scratch_types=[
            pltpu.VMEM((3, ch), jnp.float32),
            pltpu.VMEM((VD1, ch), jnp.float32),
            pltpu.VMEM((TR * VD1,), jnp.float32),
        ],
    )


def _run(lattice_py, positions, values, ch=512):
    n = positions.shape[0]
    m = lattice_py.shape[0]
    totchunks = n // ch
    pos3 = positions.T.reshape(3, totchunks, ch).transpose(1, 0, 2)
    hom3 = (
        jnp.concatenate([values, jnp.ones((n, 1), values.dtype)], axis=1)
        .T.reshape(VD1, totchunks, ch)
        .transpose(1, 0, 2)
    )
    out = _build(n, m, ch)(lattice_py.reshape(-1), pos3, hom3)
    return out.reshape(m, VD1)


@jax.jit
def kernel(lattice_py, positions, values):
    return _run(lattice_py, positions, values)


# speedup vs baseline: 1.8147x; 1.8147x over previous
import functools
import itertools

import jax
import jax.numpy as jnp
from jax import lax
from jax.experimental import pallas as pl
from jax.experimental.pallas import tpu as pltpu
from jax.experimental.pallas import tpu_sc as plsc

P0, P1, P2 = 73856093, 19349663, 83492791
NC, NS, L = 2, 16, 16
VD1 = 9
TR = 4096


@functools.cache
def _build(n, m, ch):
    nw = NC * NS
    sweeps = m // (TR * nw)
    assert sweeps * TR * nw == m
    totchunks = n // ch
    groups = ch // L

    def body(lat_hbm, pos_hbm, hom_hbm, out_hbm, pos_v, hom_v, accum):
        c = lax.axis_index("c")
        s = lax.axis_index("s")
        wid = s * NC + c

        def do_sweep(j, carry):
            base = (j * nw + wid) * TR
            pltpu.sync_copy(lat_hbm.at[pl.ds(base * VD1, TR * VD1)], accum)

            def do_chunk(k, carry):
                pltpu.sync_copy(pos_hbm.at[k], pos_v)
                pltpu.sync_copy(hom_hbm.at[k], hom_v)

                def do_group(g, carry):
                    col = g * L

                    def flo(f):
                        i = f.astype(jnp.int32)
                        i = jnp.where(f < i.astype(jnp.float32), i - 1, i)
                        return i, f - i.astype(jnp.float32)

                    bx, fx = flo(pos_v[0, pl.ds(col, L)])
                    by, fy = flo(pos_v[1, pl.ds(col, L)])
                    bz, fz = flo(pos_v[2, pl.ds(col, L)])
                    h0 = bx * P0 + by * P1 + bz * P2
                    basev = jnp.full((L,), base, jnp.int32)
                    corners = []
                    pops = []
                    for ox, oy, oz in itertools.product((0, 1), repeat=3):
                        cc = (ox * P0 + oy * P1 + oz * P2) & 0xFFFFFFFF
                        cc = cc - (1 << 32) if cc >= (1 << 31) else cc
                        hc = (h0 + cc) & (m - 1)
                        loc = hc - basev
                        inp = (loc >= jnp.zeros((L,), jnp.int32)) & (
                            loc < jnp.full((L,), TR, jnp.int32)
                        )
                        corners.append((ox, oy, oz, loc, inp))
                        pops.append(plsc.all_reduce_population_count(inp)[0])
                    nhit = (
                        pops[0] + pops[1] + pops[2] + pops[3]
                        + pops[4] + pops[5] + pops[6] + pops[7]
                    )

                    @pl.when(nhit > 0)
                    def _():
                        wx0 = 1.0 - fx
                        wy0 = 1.0 - fy
                        wz0 = 1.0 - fz
                        wxy = (wx0 * wy0, wx0 * fy, fx * wy0, fx * fy)
                        v = [hom_v[r, pl.ds(col, L)] for r in range(VD1)]
                        for ci, (ox, oy, oz, loc, inp) in enumerate(corners):

                            @pl.when(pops[ci] > 0)
                            def _(ox=ox, oy=oy, oz=oz, loc=loc, inp=inp):
                                locs = (loc & (TR - 1)) * VD1
                                wc = wxy[ox * 2 + oy] * (fz if oz else wz0)
                                we = jnp.where(inp, wc, 0.0)
                                for r in range(VD1):
                                    plsc.addupdate_scatter(
                                        accum, [locs + r], we * v[r]
                                    )

                    return carry

                return lax.fori_loop(0, groups, do_group, carry)

            carry = lax.fori_loop(0, totchunks, do_chunk, carry)
            pltpu.sync_copy(accum, out_hbm.at[pl.ds(base * VD1, TR * VD1)])
            return carry

        lax.fori_loop(0, sweeps, do_sweep, 0)

    mesh = plsc.VectorSubcoreMesh(
        core_axis_name="c", subcore_axis_name="s", num_cores=NC, num_subcores=NS
    )
    return pl.kernel(
        body,
        out_type=jax.ShapeDtypeStruct((m * VD1,), jnp.float32),
        mesh=mesh,
        compiler_params=pltpu.CompilerParams(needs_layout_passes=False),
        scratch_types=[
            pltpu.VMEM((3, ch), jnp.float32),
            pltpu.VMEM((VD1, ch), jnp.float32),
            pltpu.VMEM((TR * VD1,), jnp.float32),
        ],
    )


def _run(lattice_py, positions, values, ch=2048):
    n = positions.shape[0]
    m = lattice_py.shape[0]
    totchunks = n // ch
    pos3 = positions.T.reshape(3, totchunks, ch).transpose(1, 0, 2)
    hom3 = (
        jnp.concatenate([values, jnp.ones((n, 1), values.dtype)], axis=1)
        .T.reshape(VD1, totchunks, ch)
        .transpose(1, 0, 2)
    )
    out = _build(n, m, ch)(lattice_py.reshape(-1), pos3, hom3)
    return out.reshape(m, VD1)


@jax.jit
def kernel(lattice_py, positions, values):
    return _run(lattice_py, positions, values)


# speedup vs baseline: 2.7850x; 1.5347x over previous
import functools
import itertools

import jax
import jax.numpy as jnp
from jax import lax
from jax.experimental import pallas as pl
from jax.experimental.pallas import tpu as pltpu
from jax.experimental.pallas import tpu_sc as plsc

P0, P1, P2 = 73856093, 19349663, 83492791
NC, NS, L = 2, 16, 16
VD1 = 9
TR = 8192


@functools.cache
def _build(n, m, ch):
    nw = NC * NS
    sweeps = m // (TR * nw)
    assert sweeps * TR * nw == m
    totchunks = n // ch
    groups = ch // L

    def body(lat_hbm, pos_hbm, hom_hbm, out_hbm, pos_v, hom_v, accum):
        c = lax.axis_index("c")
        s = lax.axis_index("s")
        wid = s * NC + c

        def do_sweep(j, carry):
            base = (j * nw + wid) * TR
            pltpu.sync_copy(lat_hbm.at[pl.ds(base * VD1, TR * VD1)], accum)

            def do_chunk(k, carry):
                pltpu.sync_copy(pos_hbm.at[k], pos_v)
                pltpu.sync_copy(hom_hbm.at[k], hom_v)

                def do_group(g, carry):
                    col = g * L

                    def flo(f):
                        i = f.astype(jnp.int32)
                        i = jnp.where(f < i.astype(jnp.float32), i - 1, i)
                        return i, f - i.astype(jnp.float32)

                    bx, fx = flo(pos_v[0, pl.ds(col, L)])
                    by, fy = flo(pos_v[1, pl.ds(col, L)])
                    bz, fz = flo(pos_v[2, pl.ds(col, L)])
                    h0 = bx * P0 + by * P1 + bz * P2
                    basev = jnp.full((L,), base, jnp.int32)
                    corners = []
                    for ox, oy, oz in itertools.product((0, 1), repeat=3):
                        cc = (ox * P0 + oy * P1 + oz * P2) & 0xFFFFFFFF
                        cc = cc - (1 << 32) if cc >= (1 << 31) else cc
                        hc = (h0 + cc) & (m - 1)
                        loc = hc - basev
                        inp = (loc >= jnp.zeros((L,), jnp.int32)) & (
                            loc < jnp.full((L,), TR, jnp.int32)
                        )
                        corners.append((ox, oy, oz, loc, inp))
                    anyhit = (
                        corners[0][4] | corners[1][4] | corners[2][4]
                        | corners[3][4] | corners[4][4] | corners[5][4]
                        | corners[6][4] | corners[7][4]
                    )
                    nhit = plsc.all_reduce_population_count(anyhit)[0]

                    @pl.when(nhit > 0)
                    def _():
                        wx0 = 1.0 - fx
                        wy0 = 1.0 - fy
                        wz0 = 1.0 - fz
                        wxy = (wx0 * wy0, wx0 * fy, fx * wy0, fx * fy)
                        v = [hom_v[r, pl.ds(col, L)] for r in range(VD1)]
                        for ci, (ox, oy, oz, loc, inp) in enumerate(corners):
                            pop_c = plsc.all_reduce_population_count(inp)[0]

                            @pl.when(pop_c > 0)
                            def _(ox=ox, oy=oy, oz=oz, loc=loc, inp=inp):
                                locs = (loc & (TR - 1)) * VD1
                                wc = wxy[ox * 2 + oy] * (fz if oz else wz0)
                                we = jnp.where(inp, wc, 0.0)
                                for r in range(VD1):
                                    plsc.addupdate_scatter(
                                        accum, [locs + r], we * v[r]
                                    )

                    return carry

                return lax.fori_loop(0, groups, do_group, carry)

            carry = lax.fori_loop(0, totchunks, do_chunk, carry)
            pltpu.sync_copy(accum, out_hbm.at[pl.ds(base * VD1, TR * VD1)])
            return carry

        lax.fori_loop(0, sweeps, do_sweep, 0)

    mesh = plsc.VectorSubcoreMesh(
        core_axis_name="c", subcore_axis_name="s", num_cores=NC, num_subcores=NS
    )
    return pl.kernel(
        body,
        out_type=jax.ShapeDtypeStruct((m * VD1,), jnp.float32),
        mesh=mesh,
        compiler_params=pltpu.CompilerParams(needs_layout_passes=False),
        scratch_types=[
            pltpu.VMEM((3, ch), jnp.float32),
            pltpu.VMEM((VD1, ch), jnp.float32),
            pltpu.VMEM((TR * VD1,), jnp.float32),
        ],
    )


def _run(lattice_py, positions, values, ch=1024):
    n = positions.shape[0]
    m = lattice_py.shape[0]
    totchunks = n // ch
    pos3 = positions.T.reshape(3, totchunks, ch).transpose(1, 0, 2)
    hom3 = (
        jnp.concatenate([values, jnp.ones((n, 1), values.dtype)], axis=1)
        .T.reshape(VD1, totchunks, ch)
        .transpose(1, 0, 2)
    )
    out = _build(n, m, ch)(lattice_py.reshape(-1), pos3, hom3)
    return out.reshape(m, VD1)


@jax.jit
def kernel(lattice_py, positions, values):
    return _run(lattice_py, positions, values)
